# initial kernel scaffold (unmeasured)
import jax
import jax.numpy as jnp
from jax import lax
from jax.experimental import pallas as pl
from jax.experimental.pallas import tpu as pltpu

N_DEV = 8


def kernel(x, w_mat, scale_x, scale_w):
    m, k_per = x.shape
    _, n = w_mat.shape
    m_chunk = m // N_DEV

    def body(x_ref, w_ref, sx_ref, sw_ref, out_ref,
             sbuf, rbuf, send_sem, recv_sem, ready_sem):
        my = lax.axis_index("i")
        left = lax.rem(my + N_DEV - 1, N_DEV)
        right = lax.rem(my + 1, N_DEV)

        barrier = pltpu.get_barrier_semaphore()
        for nbr in (left, right):
            pl.semaphore_signal(barrier, inc=1, device_id=(nbr,),
                                device_id_type=pl.DeviceIdType.MESH)
        pl.semaphore_wait(barrier, 2)

        scale = sx_ref[0] * sw_ref[0]
        w_bf = w_ref[...].astype(jnp.bfloat16)

        def chunk_mm(c):
            xs = x_ref[pl.ds(c * m_chunk, m_chunk), :].astype(jnp.bfloat16)
            return jnp.dot(xs, w_bf, preferred_element_type=jnp.float32)

        def hop():
            pl.semaphore_signal(ready_sem, inc=1, device_id=(left,),
                                device_id_type=pl.DeviceIdType.MESH)
            pl.semaphore_wait(ready_sem, 1)
            rdma = pltpu.make_async_remote_copy(
                src_ref=sbuf, dst_ref=rbuf,
                send_sem=send_sem, recv_sem=recv_sem,
                device_id=(right,), device_id_type=pl.DeviceIdType.MESH)
            rdma.start()
            rdma.wait()

        sbuf[...] = chunk_mm(my)
        for s in range(N_DEV - 1):
            hop()
            c = lax.rem(my - s - 1 + 2 * N_DEV, N_DEV)
            acc = rbuf[...] + chunk_mm(c)
            if s < N_DEV - 2:
                sbuf[...] = acc
            else:
                final = jnp.maximum(acc * scale, 0.0)
                out_ref[pl.ds(c * m_chunk, m_chunk), :] = final
                sbuf[...] = final

        for t in range(N_DEV - 1):
            hop()
            c = lax.rem(my - t + 2 * N_DEV, N_DEV)
            got = rbuf[...]
            out_ref[pl.ds(c * m_chunk, m_chunk), :] = got
            if t < N_DEV - 2:
                sbuf[...] = got

    return pl.pallas_call(
        body,
        out_shape=jax.ShapeDtypeStruct((m, n), jnp.float32),
        in_specs=[
            pl.BlockSpec(memory_space=pltpu.VMEM),
            pl.BlockSpec(memory_space=pltpu.VMEM),
            pl.BlockSpec(memory_space=pltpu.SMEM),
            pl.BlockSpec(memory_space=pltpu.SMEM),
        ],
        out_specs=pl.BlockSpec(memory_space=pltpu.VMEM),
        scratch_shapes=[
            pltpu.VMEM((m_chunk, n), jnp.float32),
            pltpu.VMEM((m_chunk, n), jnp.float32),
            pltpu.SemaphoreType.DMA,
            pltpu.SemaphoreType.DMA,
            pltpu.SemaphoreType.REGULAR,
        ],
        compiler_params=pltpu.CompilerParams(collective_id=0),
    )(x, w_mat, scale_x, scale_w)


# baseline (device time: 733183 ns/iter reference)
import jax
import jax.numpy as jnp
from jax import lax
from jax.experimental import pallas as pl
from jax.experimental.pallas import tpu as pltpu

N_DEV = 8


def kernel(x, w_mat, scale_x, scale_w):
    m, k_per = x.shape
    _, n = w_mat.shape
    m_chunk = m // N_DEV

    def body(x_ref, w_ref, sx_ref, sw_ref, out_ref,
             sbuf, rbuf, send_sem, recv_sem, ready_sem):
        my = lax.axis_index("i")
        left = lax.rem(my + N_DEV - 1, N_DEV)
        right = lax.rem(my + 1, N_DEV)

        barrier = pltpu.get_barrier_semaphore()
        for nbr in (left, right):
            pl.semaphore_signal(barrier, inc=1, device_id=(nbr,),
                                device_id_type=pl.DeviceIdType.MESH)
        pl.semaphore_wait(barrier, 2)

        scale = sx_ref[0] * sw_ref[0]
        w_bf = w_ref[...].astype(jnp.bfloat16)

        def chunk_mm(c):
            xs = x_ref[pl.ds(c * m_chunk, m_chunk), :].astype(jnp.bfloat16)
            return jnp.dot(xs, w_bf, preferred_element_type=jnp.float32)

        def hop():
            pl.semaphore_signal(ready_sem, inc=1, device_id=(left,),
                                device_id_type=pl.DeviceIdType.MESH)
            pl.semaphore_wait(ready_sem, 1)
            rdma = pltpu.make_async_remote_copy(
                src_ref=sbuf, dst_ref=rbuf,
                send_sem=send_sem, recv_sem=recv_sem,
                device_id=(right,), device_id_type=pl.DeviceIdType.MESH)
            rdma.start()
            rdma.wait()

        sbuf[...] = chunk_mm(my)
        for s in range(N_DEV - 1):
            hop()
            c = lax.rem(my - s - 1 + 2 * N_DEV, N_DEV)
            acc = rbuf[...] + chunk_mm(c)
            if s < N_DEV - 2:
                sbuf[...] = acc
            else:
                final = jnp.maximum(acc * scale, 0.0)
                out_ref[pl.ds(c * m_chunk, m_chunk), :] = final
                sbuf[...] = final

        for t in range(N_DEV - 1):
            hop()
            c = lax.rem(my - t + 2 * N_DEV, N_DEV)
            got = rbuf[...]
            out_ref[pl.ds(c * m_chunk, m_chunk), :] = got
            if t < N_DEV - 2:
                sbuf[...] = got

    return pl.pallas_call(
        body,
        out_shape=jax.ShapeDtypeStruct((m, n), jnp.float32),
        in_specs=[
            pl.BlockSpec(memory_space=pltpu.VMEM),
            pl.BlockSpec(memory_space=pltpu.VMEM),
            pl.BlockSpec(memory_space=pltpu.SMEM),
            pl.BlockSpec(memory_space=pltpu.SMEM),
        ],
        out_specs=pl.BlockSpec(memory_space=pltpu.VMEM),
        scratch_shapes=[
            pltpu.VMEM((m_chunk, n), jnp.float32),
            pltpu.VMEM((m_chunk, n), jnp.float32),
            pltpu.SemaphoreType.DMA,
            pltpu.SemaphoreType.DMA,
            pltpu.SemaphoreType.REGULAR,
        ],
        compiler_params=pltpu.CompilerParams(
            collective_id=0, vmem_limit_bytes=100 * 1024 * 1024),
    )(x, w_mat, scale_x, scale_w)


# device time: 253365 ns/iter; 2.8938x vs baseline; 2.8938x over previous
import jax
import jax.numpy as jnp
from jax import lax
from jax.experimental import pallas as pl
from jax.experimental.pallas import tpu as pltpu

N_DEV = 8
MESH = pl.DeviceIdType.MESH


def kernel(x, w_mat, scale_x, scale_w):
    m, k_per = x.shape
    _, n = w_mat.shape
    mc = m // N_DEV
    hn = n // 2

    def body(x_ref, w_ref, sx_ref, sw_ref, out_ref,
             fbuf, bbuf, fsend, frecv, bsend, brecv, fready, bready):
        my = lax.axis_index("i")
        left = lax.rem(my + N_DEV - 1, N_DEV)
        right = lax.rem(my + 1, N_DEV)

        barrier = pltpu.get_barrier_semaphore()
        for nbr in (left, right):
            pl.semaphore_signal(barrier, inc=1, device_id=(nbr,),
                                device_id_type=MESH)
        pl.semaphore_wait(barrier, 2)

        scale = sx_ref[0] * sw_ref[0]

        def mm(c, lo):
            xs = x_ref[pl.ds(c * mc, mc), :].astype(jnp.bfloat16)
            wb = w_ref[:, lo:lo + hn].astype(jnp.bfloat16)
            return jnp.dot(xs, wb, preferred_element_type=jnp.float32)

        def hop(h, compute_fn):
            pl.semaphore_signal(fready, inc=1, device_id=(left,),
                                device_id_type=MESH)
            pl.semaphore_signal(bready, inc=1, device_id=(right,),
                                device_id_type=MESH)
            pl.semaphore_wait(fready, 1)
            pl.semaphore_wait(bready, 1)
            rf = pltpu.make_async_remote_copy(
                src_ref=fbuf.at[h % 2], dst_ref=fbuf.at[(h + 1) % 2],
                send_sem=fsend, recv_sem=frecv,
                device_id=(right,), device_id_type=MESH)
            rb = pltpu.make_async_remote_copy(
                src_ref=bbuf.at[h % 2], dst_ref=bbuf.at[(h + 1) % 2],
                send_sem=bsend, recv_sem=brecv,
                device_id=(left,), device_id_type=MESH)
            rf.start()
            rb.start()
            res = compute_fn()
            rf.wait()
            rb.wait()
            return res

        fbuf[0] = mm(my, 0).astype(jnp.bfloat16)
        bbuf[0] = mm(my, hn).astype(jnp.bfloat16)
        h = 0
        for s in range(N_DEV - 1):
            cf = lax.rem(my - s - 1 + 2 * N_DEV, N_DEV)
            cb = lax.rem(my + s + 1, N_DEV)
            mmf, mmb = hop(h, lambda cf=cf, cb=cb: (mm(cf, 0), mm(cb, hn)))
            r = (h + 1) % 2
            accf = fbuf[r].astype(jnp.float32) + mmf
            accb = bbuf[r].astype(jnp.float32) + mmb
            if s < N_DEV - 2:
                fbuf[r] = accf.astype(jnp.bfloat16)
                bbuf[r] = accb.astype(jnp.bfloat16)
            else:
                finf = jnp.maximum(accf * scale, 0.0)
                finb = jnp.maximum(accb * scale, 0.0)
                out_ref[pl.ds(cf * mc, mc), 0:hn] = finf
                out_ref[pl.ds(cb * mc, mc), hn:n] = finb
                fbuf[r] = finf.astype(jnp.bfloat16)
                bbuf[r] = finb.astype(jnp.bfloat16)
            h += 1

        for t in range(N_DEV - 1):
            hop(h, lambda: None)
            r = (h + 1) % 2
            cf = lax.rem(my - t + 2 * N_DEV, N_DEV)
            cb = lax.rem(my + t, N_DEV)
            out_ref[pl.ds(cf * mc, mc), 0:hn] = fbuf[r].astype(jnp.float32)
            out_ref[pl.ds(cb * mc, mc), hn:n] = bbuf[r].astype(jnp.float32)
            h += 1

    return pl.pallas_call(
        body,
        out_shape=jax.ShapeDtypeStruct((m, n), jnp.float32),
        in_specs=[
            pl.BlockSpec(memory_space=pltpu.VMEM),
            pl.BlockSpec(memory_space=pltpu.VMEM),
            pl.BlockSpec(memory_space=pltpu.SMEM),
            pl.BlockSpec(memory_space=pltpu.SMEM),
        ],
        out_specs=pl.BlockSpec(memory_space=pltpu.VMEM),
        scratch_shapes=[
            pltpu.VMEM((2, mc, hn), jnp.bfloat16),
            pltpu.VMEM((2, mc, hn), jnp.bfloat16),
            pltpu.SemaphoreType.DMA,
            pltpu.SemaphoreType.DMA,
            pltpu.SemaphoreType.DMA,
            pltpu.SemaphoreType.DMA,
            pltpu.SemaphoreType.REGULAR,
            pltpu.SemaphoreType.REGULAR,
        ],
        compiler_params=pltpu.CompilerParams(
            collective_id=0, vmem_limit_bytes=100 * 1024 * 1024),
    )(x, w_mat, scale_x, scale_w)


# device time: 243980 ns/iter; 3.0051x vs baseline; 1.0385x over previous
import jax
import jax.numpy as jnp
from jax import lax
from jax.experimental import pallas as pl
from jax.experimental.pallas import tpu as pltpu

N_DEV = 8
MESH = pl.DeviceIdType.MESH
_RING = [0, 1, 2, 3, 7, 6, 5, 4]


def kernel(x, w_mat, scale_x, scale_w):
    m, k_per = x.shape
    _, n = w_mat.shape
    mc = m // N_DEV
    hn = n // 2

    ring = jnp.asarray(_RING, dtype=jnp.int32)
    my = lax.axis_index("i")
    p = ring[my]
    right_d = ring[lax.rem(p + 1, N_DEV)]
    left_d = ring[lax.rem(p + N_DEV - 1, N_DEV)]
    pos = jnp.reshape(p, (1,))
    left_a = jnp.reshape(left_d, (1,))
    right_a = jnp.reshape(right_d, (1,))

    def body(pos_ref, left_ref, right_ref, x_ref, w_ref, sx_ref, sw_ref,
             out_ref, fbuf, bbuf, fsend, frecv, bsend, brecv,
             fready, bready):
        p = pos_ref[0]
        left = left_ref[0]
        right = right_ref[0]

        barrier = pltpu.get_barrier_semaphore()
        for nbr in (left, right):
            pl.semaphore_signal(barrier, inc=1, device_id=(nbr,),
                                device_id_type=MESH)
        pl.semaphore_wait(barrier, 2)

        scale = sx_ref[0] * sw_ref[0]

        def mm(c, lo):
            xs = x_ref[pl.ds(c * mc, mc), :].astype(jnp.bfloat16)
            wb = w_ref[:, lo:lo + hn].astype(jnp.bfloat16)
            return jnp.dot(xs, wb, preferred_element_type=jnp.float32)

        def hop(h, compute_fn):
            pl.semaphore_signal(fready, inc=1, device_id=(left,),
                                device_id_type=MESH)
            pl.semaphore_signal(bready, inc=1, device_id=(right,),
                                device_id_type=MESH)
            pl.semaphore_wait(fready, 1)
            pl.semaphore_wait(bready, 1)
            rf = pltpu.make_async_remote_copy(
                src_ref=fbuf.at[h % 2], dst_ref=fbuf.at[(h + 1) % 2],
                send_sem=fsend, recv_sem=frecv,
                device_id=(right,), device_id_type=MESH)
            rb = pltpu.make_async_remote_copy(
                src_ref=bbuf.at[h % 2], dst_ref=bbuf.at[(h + 1) % 2],
                send_sem=bsend, recv_sem=brecv,
                device_id=(left,), device_id_type=MESH)
            rf.start()
            rb.start()
            res = compute_fn()
            rf.wait()
            rb.wait()
            return res

        fbuf[0] = mm(p, 0).astype(jnp.bfloat16)
        bbuf[0] = mm(p, hn).astype(jnp.bfloat16)
        h = 0
        cf = cb = None
        finf = finb = None
        for s in range(N_DEV - 1):
            cf = lax.rem(p - s - 1 + 2 * N_DEV, N_DEV)
            cb = lax.rem(p + s + 1, N_DEV)
            mmf, mmb = hop(h, lambda cf=cf, cb=cb: (mm(cf, 0), mm(cb, hn)))
            r = (h + 1) % 2
            accf = fbuf[r].astype(jnp.float32) + mmf
            accb = bbuf[r].astype(jnp.float32) + mmb
            if s < N_DEV - 2:
                fbuf[r] = accf.astype(jnp.bfloat16)
                bbuf[r] = accb.astype(jnp.bfloat16)
            else:
                finf = jnp.maximum(accf * scale, 0.0)
                finb = jnp.maximum(accb * scale, 0.0)
                fbuf[r] = finf.astype(jnp.bfloat16)
                bbuf[r] = finb.astype(jnp.bfloat16)
            h += 1

        pending = (cf, finf, cb, finb)
        for t in range(N_DEV - 1):
            def store_prev(pending=pending):
                c0, vf, c1, vb = pending
                out_ref[pl.ds(c0 * mc, mc), 0:hn] = vf
                out_ref[pl.ds(c1 * mc, mc), hn:n] = vb

            hop(h, store_prev)
            r = (h + 1) % 2
            cf = lax.rem(p - t + 2 * N_DEV, N_DEV)
            cb = lax.rem(p + t, N_DEV)
            pending = (cf, fbuf[r].astype(jnp.float32),
                       cb, bbuf[r].astype(jnp.float32))
            h += 1
        c0, vf, c1, vb = pending
        out_ref[pl.ds(c0 * mc, mc), 0:hn] = vf
        out_ref[pl.ds(c1 * mc, mc), hn:n] = vb

    return pl.pallas_call(
        body,
        out_shape=jax.ShapeDtypeStruct((m, n), jnp.float32),
        in_specs=[
            pl.BlockSpec(memory_space=pltpu.SMEM),
            pl.BlockSpec(memory_space=pltpu.SMEM),
            pl.BlockSpec(memory_space=pltpu.SMEM),
            pl.BlockSpec(memory_space=pltpu.VMEM),
            pl.BlockSpec(memory_space=pltpu.VMEM),
            pl.BlockSpec(memory_space=pltpu.SMEM),
            pl.BlockSpec(memory_space=pltpu.SMEM),
        ],
        out_specs=pl.BlockSpec(memory_space=pltpu.VMEM),
        scratch_shapes=[
            pltpu.VMEM((2, mc, hn), jnp.bfloat16),
            pltpu.VMEM((2, mc, hn), jnp.bfloat16),
            pltpu.SemaphoreType.DMA,
            pltpu.SemaphoreType.DMA,
            pltpu.SemaphoreType.DMA,
            pltpu.SemaphoreType.DMA,
            pltpu.SemaphoreType.REGULAR,
            pltpu.SemaphoreType.REGULAR,
        ],
        compiler_params=pltpu.CompilerParams(
            collective_id=0, vmem_limit_bytes=100 * 1024 * 1024),
    )(pos, left_a, right_a, x, w_mat, scale_x, scale_w)


# device time: 212536 ns/iter; 3.4497x vs baseline; 1.1479x over previous
import jax
import jax.numpy as jnp
from jax import lax
from jax.experimental import pallas as pl
from jax.experimental.pallas import tpu as pltpu

N_DEV = 8
MESH = pl.DeviceIdType.MESH
N_STREAMS = 4
_RING = [0, 1, 2, 3, 7, 6, 5, 4]


def kernel(x, w_mat, scale_x, scale_w):
    m, k_per = x.shape
    _, n = w_mat.shape
    mc = m // N_DEV
    qn = n // N_STREAMS

    ring = jnp.asarray(_RING, dtype=jnp.int32)
    my = lax.axis_index("i")
    p = ring[my]
    right_d = ring[lax.rem(p + 1, N_DEV)]
    left_d = ring[lax.rem(p + N_DEV - 1, N_DEV)]
    pos = jnp.reshape(p, (1,))
    left_a = jnp.reshape(left_d, (1,))
    right_a = jnp.reshape(right_d, (1,))

    n_hops = 2 * (N_DEV - 1)
    rs_last = N_DEV - 2

    def body(pos_ref, left_ref, right_ref, x_ref, w_ref, sx_ref, sw_ref,
             out_ref, bufs, ssend, srecv, scred):
        p = pos_ref[0]
        left = left_ref[0]
        right = right_ref[0]

        barrier = pltpu.get_barrier_semaphore()
        for nbr in (left, right):
            pl.semaphore_signal(barrier, inc=1, device_id=(nbr,),
                                device_id_type=MESH)
        pl.semaphore_wait(barrier, 2)

        scale = sx_ref[0] * sw_ref[0]

        los = [0, qn, 2 * qn, 3 * qn]
        is_fwd = [True, True, False, False]

        def receiver(k):
            return right if is_fwd[k] else left

        def sender(k):
            return left if is_fwd[k] else right

        def mm(c, lo):
            xs = x_ref[pl.ds(c * mc, mc), :].astype(jnp.bfloat16)
            wb = w_ref[:, lo:lo + qn].astype(jnp.bfloat16)
            return jnp.dot(xs, wb, preferred_element_type=jnp.float32)

        def chunk_rs(k, s):
            if is_fwd[k]:
                return lax.rem(p - s - 1 + 2 * N_DEV, N_DEV)
            return lax.rem(p + s + 1, N_DEV)

        def chunk_ag(k, t):
            if is_fwd[k]:
                return lax.rem(p - t + 2 * N_DEV, N_DEV)
            return lax.rem(p + t, N_DEV)

        def launch(k, h):
            pl.semaphore_signal(scred.at[k], inc=1, device_id=(sender(k),),
                                device_id_type=MESH)
            pl.semaphore_wait(scred.at[k], 1)
            r = pltpu.make_async_remote_copy(
                src_ref=bufs.at[k, h % 2], dst_ref=bufs.at[k, (h + 1) % 2],
                send_sem=ssend.at[k], recv_sem=srecv.at[k],
                device_id=(receiver(k),), device_id_type=MESH)
            r.start()
            return r

        for k in range(N_STREAMS):
            bufs[k, 0] = mm(p, los[k]).astype(jnp.bfloat16)
        inflight = [launch(k, 0) for k in range(N_STREAMS)]

        for h in range(n_hops):
            for k in range(N_STREAMS):
                lo = los[k]
                if h <= rs_last:
                    part = mm(chunk_rs(k, h), lo)
                inflight[k].wait()
                r = (h + 1) % 2
                if h < rs_last:
                    acc = bufs[k, r].astype(jnp.float32) + part
                    bufs[k, r] = acc.astype(jnp.bfloat16)
                    inflight[k] = launch(k, h + 1)
                elif h == rs_last:
                    acc = bufs[k, r].astype(jnp.float32) + part
                    fin = jnp.maximum(acc * scale, 0.0)
                    bufs[k, r] = fin.astype(jnp.bfloat16)
                    inflight[k] = launch(k, h + 1)
                    c = chunk_rs(k, h)
                    out_ref[pl.ds(c * mc, mc), lo:lo + qn] = fin
                else:
                    if h < n_hops - 1:
                        inflight[k] = launch(k, h + 1)
                    c = chunk_ag(k, h - (rs_last + 1))
                    out_ref[pl.ds(c * mc, mc), lo:lo + qn] = (
                        bufs[k, r].astype(jnp.float32))

    return pl.pallas_call(
        body,
        out_shape=jax.ShapeDtypeStruct((m, n), jnp.float32),
        in_specs=[
            pl.BlockSpec(memory_space=pltpu.SMEM),
            pl.BlockSpec(memory_space=pltpu.SMEM),
            pl.BlockSpec(memory_space=pltpu.SMEM),
            pl.BlockSpec(memory_space=pltpu.VMEM),
            pl.BlockSpec(memory_space=pltpu.VMEM),
            pl.BlockSpec(memory_space=pltpu.SMEM),
            pl.BlockSpec(memory_space=pltpu.SMEM),
        ],
        out_specs=pl.BlockSpec(memory_space=pltpu.VMEM),
        scratch_shapes=[
            pltpu.VMEM((N_STREAMS, 2, mc, qn), jnp.bfloat16),
            pltpu.SemaphoreType.DMA((N_STREAMS,)),
            pltpu.SemaphoreType.DMA((N_STREAMS,)),
            pltpu.SemaphoreType.REGULAR((N_STREAMS,)),
        ],
        compiler_params=pltpu.CompilerParams(
            collective_id=0, vmem_limit_bytes=100 * 1024 * 1024),
    )(pos, left_a, right_a, x, w_mat, scale_x, scale_w)


# device time: 193054 ns/iter; 3.7978x vs baseline; 1.1009x over previous
import jax
import jax.numpy as jnp
from jax import lax
from jax.experimental import pallas as pl
from jax.experimental.pallas import tpu as pltpu

N_DEV = 8
MESH = pl.DeviceIdType.MESH
N_STREAMS = 4


def kernel(x, w_mat, scale_x, scale_w):
    m, k_per = x.shape
    _, n = w_mat.shape
    mc = m // N_DEV
    qn = n // N_STREAMS

    n_hops = 2 * (N_DEV - 1)
    rs_last = N_DEV - 2

    def ring_id(q):
        return jnp.where(q < 4, q, 11 - q)

    def body(x_ref, w_ref, sx_ref, sw_ref,
             out_ref, bufs, ssend, srecv, scred):
        my = lax.axis_index("i")
        p = ring_id(my)
        right = ring_id(lax.rem(p + 1, N_DEV))
        left = ring_id(lax.rem(p + N_DEV - 1, N_DEV))

        barrier = pltpu.get_barrier_semaphore()
        for nbr in (left, right):
            pl.semaphore_signal(barrier, inc=1, device_id=(nbr,),
                                device_id_type=MESH)
        pl.semaphore_wait(barrier, 2)

        scale = sx_ref[0] * sw_ref[0]

        los = [0, qn, 2 * qn, 3 * qn]
        is_fwd = [True, True, False, False]

        def receiver(k):
            return right if is_fwd[k] else left

        def sender(k):
            return left if is_fwd[k] else right

        def mm(c, lo):
            xs = x_ref[pl.ds(c * mc, mc), :].astype(jnp.bfloat16)
            wb = w_ref[:, lo:lo + qn].astype(jnp.bfloat16)
            return jnp.dot(xs, wb, preferred_element_type=jnp.float32)

        def chunk_rs(k, s):
            if is_fwd[k]:
                return lax.rem(p - s - 1 + 2 * N_DEV, N_DEV)
            return lax.rem(p + s + 1, N_DEV)

        def chunk_ag(k, t):
            if is_fwd[k]:
                return lax.rem(p - t + 2 * N_DEV, N_DEV)
            return lax.rem(p + t, N_DEV)

        def grant(k):
            pl.semaphore_signal(scred.at[k], inc=1, device_id=(sender(k),),
                                device_id_type=MESH)

        def fire(k, h):
            pl.semaphore_wait(scred.at[k], 1)
            r = pltpu.make_async_remote_copy(
                src_ref=bufs.at[k, h % 2], dst_ref=bufs.at[k, (h + 1) % 2],
                send_sem=ssend.at[k], recv_sem=srecv.at[k],
                device_id=(receiver(k),), device_id_type=MESH)
            r.start()
            return r

        for k in range(N_STREAMS):
            bufs[k, 0] = mm(p, los[k]).astype(jnp.bfloat16)
        for k in range(N_STREAMS):
            grant(k)
        inflight = [fire(k, 0) for k in range(N_STREAMS)]

        for h in range(n_hops):
            for k in range(N_STREAMS):
                lo = los[k]
                if h <= rs_last:
                    part = mm(chunk_rs(k, h), lo)
                inflight[k].wait()
                if h < n_hops - 1:
                    grant(k)
                r = (h + 1) % 2
                if h < rs_last:
                    acc = bufs[k, r].astype(jnp.float32) + part
                    bufs[k, r] = acc.astype(jnp.bfloat16)
                    inflight[k] = fire(k, h + 1)
                elif h == rs_last:
                    acc = bufs[k, r].astype(jnp.float32) + part
                    fin = jnp.maximum(acc * scale, 0.0).astype(jnp.bfloat16)
                    bufs[k, r] = fin
                    inflight[k] = fire(k, h + 1)
                    c = chunk_rs(k, h)
                    out_ref[pl.ds(c * mc, mc), lo:lo + qn] = fin
                else:
                    if h < n_hops - 1:
                        inflight[k] = fire(k, h + 1)
                    c = chunk_ag(k, h - (rs_last + 1))
                    out_ref[pl.ds(c * mc, mc), lo:lo + qn] = bufs[k, r]

    return pl.pallas_call(
        body,
        out_shape=jax.ShapeDtypeStruct((m, n), jnp.bfloat16),
        in_specs=[
            pl.BlockSpec(memory_space=pltpu.VMEM),
            pl.BlockSpec(memory_space=pltpu.VMEM),
            pl.BlockSpec(memory_space=pltpu.SMEM),
            pl.BlockSpec(memory_space=pltpu.SMEM),
        ],
        out_specs=pl.BlockSpec(memory_space=pltpu.VMEM),
        scratch_shapes=[
            pltpu.VMEM((N_STREAMS, 2, mc, qn), jnp.bfloat16),
            pltpu.SemaphoreType.DMA((N_STREAMS,)),
            pltpu.SemaphoreType.DMA((N_STREAMS,)),
            pltpu.SemaphoreType.REGULAR((N_STREAMS,)),
        ],
        compiler_params=pltpu.CompilerParams(
            collective_id=0, vmem_limit_bytes=100 * 1024 * 1024),
    )(x, w_mat, scale_x, scale_w)


# device time: 188095 ns/iter; 3.8979x vs baseline; 1.0264x over previous
import jax
import jax.numpy as jnp
from jax import lax
from jax.experimental import pallas as pl
from jax.experimental.pallas import tpu as pltpu

N_DEV = 8
MESH = pl.DeviceIdType.MESH
N_STREAMS = 4


def kernel(x, w_mat, scale_x, scale_w):
    m, k_per = x.shape
    _, n = w_mat.shape
    mc = m // N_DEV
    qn = n // N_STREAMS

    n_hops = 2 * (N_DEV - 1)
    rs_last = N_DEV - 2

    def ring_id(q):
        return jnp.where(q < 4, q, 11 - q)

    def body(x_ref, w_ref, sx_ref, sw_ref,
             out_ref, bufs, ssend, srecv, scred, osem):
        my = lax.axis_index("i")
        p = ring_id(my)
        right = ring_id(lax.rem(p + 1, N_DEV))
        left = ring_id(lax.rem(p + N_DEV - 1, N_DEV))

        barrier = pltpu.get_barrier_semaphore()
        for nbr in (left, right):
            pl.semaphore_signal(barrier, inc=1, device_id=(nbr,),
                                device_id_type=MESH)
        pl.semaphore_wait(barrier, 2)

        scale = sx_ref[0] * sw_ref[0]

        los = [0, qn, 2 * qn, 3 * qn]
        is_fwd = [True, True, False, False]

        def receiver(k):
            return right if is_fwd[k] else left

        def sender(k):
            return left if is_fwd[k] else right

        def mm(c, lo):
            xs = x_ref[pl.ds(c * mc, mc), :].astype(jnp.bfloat16)
            wb = w_ref[:, lo:lo + qn].astype(jnp.bfloat16)
            return jnp.dot(xs, wb, preferred_element_type=jnp.float32)

        def chunk_rs(k, s):
            if is_fwd[k]:
                return lax.rem(p - s - 1 + 2 * N_DEV, N_DEV)
            return lax.rem(p + s + 1, N_DEV)

        def chunk_ag(k, t):
            if is_fwd[k]:
                return lax.rem(p - t + 2 * N_DEV, N_DEV)
            return lax.rem(p + t, N_DEV)

        def grant(k):
            pl.semaphore_signal(scred.at[k], inc=1, device_id=(sender(k),),
                                device_id_type=MESH)

        def fire(k, h):
            pl.semaphore_wait(scred.at[k], 1)
            r = pltpu.make_async_remote_copy(
                src_ref=bufs.at[k, h % 2], dst_ref=bufs.at[k, (h + 1) % 2],
                send_sem=ssend.at[k], recv_sem=srecv.at[k],
                device_id=(receiver(k),), device_id_type=MESH)
            r.start()
            return r

        def store(k, h, c):
            d = pltpu.make_async_copy(
                bufs.at[k, (h + 1) % 2],
                out_ref.at[pl.ds(c * mc, mc), pl.ds(los[k], qn)],
                osem.at[k])
            d.start()
            return d

        for k in range(N_STREAMS):
            bufs[k, 0] = mm(p, los[k]).astype(jnp.bfloat16)
        for k in range(N_STREAMS):
            grant(k)
        inflight = [fire(k, 0) for k in range(N_STREAMS)]
        outdma = [None] * N_STREAMS

        for h in range(n_hops):
            for k in range(N_STREAMS):
                lo = los[k]
                if h <= rs_last:
                    part = mm(chunk_rs(k, h), lo)
                inflight[k].wait()
                if outdma[k] is not None:
                    outdma[k].wait()
                    outdma[k] = None
                if h < n_hops - 1:
                    grant(k)
                r = (h + 1) % 2
                if h < rs_last:
                    acc = bufs[k, r].astype(jnp.float32) + part
                    bufs[k, r] = acc.astype(jnp.bfloat16)
                    inflight[k] = fire(k, h + 1)
                elif h == rs_last:
                    acc = bufs[k, r].astype(jnp.float32) + part
                    fin = jnp.maximum(acc * scale, 0.0).astype(jnp.bfloat16)
                    bufs[k, r] = fin
                    inflight[k] = fire(k, h + 1)
                    outdma[k] = store(k, h, chunk_rs(k, h))
                else:
                    if h < n_hops - 1:
                        inflight[k] = fire(k, h + 1)
                    outdma[k] = store(k, h, chunk_ag(k, h - (rs_last + 1)))

        for k in range(N_STREAMS):
            outdma[k].wait()

    return pl.pallas_call(
        body,
        out_shape=jax.ShapeDtypeStruct((m, n), jnp.bfloat16),
        in_specs=[
            pl.BlockSpec(memory_space=pltpu.VMEM),
            pl.BlockSpec(memory_space=pltpu.VMEM),
            pl.BlockSpec(memory_space=pltpu.SMEM),
            pl.BlockSpec(memory_space=pltpu.SMEM),
        ],
        out_specs=pl.BlockSpec(memory_space=pl.ANY),
        scratch_shapes=[
            pltpu.VMEM((N_STREAMS, 2, mc, qn), jnp.bfloat16),
            pltpu.SemaphoreType.DMA((N_STREAMS,)),
            pltpu.SemaphoreType.DMA((N_STREAMS,)),
            pltpu.SemaphoreType.REGULAR((N_STREAMS,)),
            pltpu.SemaphoreType.DMA((N_STREAMS,)),
        ],
        compiler_params=pltpu.CompilerParams(
            collective_id=0, vmem_limit_bytes=100 * 1024 * 1024),
    )(x, w_mat, scale_x, scale_w)


# device time: 186775 ns/iter; 3.9255x vs baseline; 1.0071x over previous
import jax
import jax.numpy as jnp
from jax import lax
from jax.experimental import pallas as pl
from jax.experimental.pallas import tpu as pltpu

N_DEV = 8
MESH = pl.DeviceIdType.MESH
N_STREAMS = 4


def kernel(x, w_mat, scale_x, scale_w):
    m, k_per = x.shape
    _, n = w_mat.shape
    mc = m // N_DEV
    qn = n // N_STREAMS

    n_hops = 2 * (N_DEV - 1)
    rs_last = N_DEV - 2

    def ring_id(q):
        return jnp.where(q < 4, q, 11 - q)

    def body(x_ref, w_ref, sx_ref, sw_ref,
             out_ref, bufs, ssend, srecv, scred, osem):
        my = lax.axis_index("i")
        p = ring_id(my)
        right = ring_id(lax.rem(p + 1, N_DEV))
        left = ring_id(lax.rem(p + N_DEV - 1, N_DEV))

        barrier = pltpu.get_barrier_semaphore()
        for nbr in (left, right):
            pl.semaphore_signal(barrier, inc=1, device_id=(nbr,),
                                device_id_type=MESH)
        pl.semaphore_wait(barrier, 2)

        scale = sx_ref[0] * sw_ref[0]

        los = [0, qn, 2 * qn, 3 * qn]
        is_fwd = [True, True, False, False]

        def receiver(k):
            return right if is_fwd[k] else left

        def sender(k):
            return left if is_fwd[k] else right

        def mm(c, lo):
            xs = x_ref[pl.ds(c * mc, mc), :].astype(jnp.bfloat16)
            wb = w_ref[:, lo:lo + qn].astype(jnp.bfloat16)
            return jnp.dot(xs, wb, preferred_element_type=jnp.float32)

        def chunk_rs(k, s):
            if is_fwd[k]:
                return lax.rem(p - s - 1 + 2 * N_DEV, N_DEV)
            return lax.rem(p + s + 1, N_DEV)

        def chunk_ag(k, t):
            if is_fwd[k]:
                return lax.rem(p - t + 2 * N_DEV, N_DEV)
            return lax.rem(p + t, N_DEV)

        def grant(k):
            pl.semaphore_signal(scred.at[k], inc=1, device_id=(sender(k),),
                                device_id_type=MESH)

        def fire(k, h):
            pl.semaphore_wait(scred.at[k], 1)
            r = pltpu.make_async_remote_copy(
                src_ref=bufs.at[k, h % 2], dst_ref=bufs.at[k, (h + 1) % 2],
                send_sem=ssend.at[k], recv_sem=srecv.at[k],
                device_id=(receiver(k),), device_id_type=MESH)
            r.start()
            return r

        def store(k, h, c):
            d = pltpu.make_async_copy(
                bufs.at[k, (h + 1) % 2],
                out_ref.at[pl.ds(c * mc, mc), pl.ds(los[k], qn)],
                osem.at[k])
            d.start()
            return d

        inflight = [None] * N_STREAMS
        for k in range(N_STREAMS):
            bufs[k, 0] = mm(p, los[k]).astype(jnp.bfloat16)
            grant(k)
            inflight[k] = fire(k, 0)
        outdma = [None] * N_STREAMS

        for h in range(n_hops):
            parts = [None] * N_STREAMS
            if h <= rs_last:
                for k in range(N_STREAMS):
                    parts[k] = mm(chunk_rs(k, h), los[k])
            for k in range(N_STREAMS):
                lo = los[k]
                part = parts[k]
                inflight[k].wait()
                if outdma[k] is not None:
                    outdma[k].wait()
                    outdma[k] = None
                if h < n_hops - 1:
                    grant(k)
                r = (h + 1) % 2
                if h < rs_last:
                    acc = bufs[k, r].astype(jnp.float32) + part
                    bufs[k, r] = acc.astype(jnp.bfloat16)
                    inflight[k] = fire(k, h + 1)
                elif h == rs_last:
                    acc = bufs[k, r].astype(jnp.float32) + part
                    fin = jnp.maximum(acc * scale, 0.0).astype(jnp.bfloat16)
                    bufs[k, r] = fin
                    inflight[k] = fire(k, h + 1)
                    outdma[k] = store(k, h, chunk_rs(k, h))
                else:
                    if h < n_hops - 1:
                        inflight[k] = fire(k, h + 1)
                    outdma[k] = store(k, h, chunk_ag(k, h - (rs_last + 1)))

        for k in range(N_STREAMS):
            outdma[k].wait()

    return pl.pallas_call(
        body,
        out_shape=jax.ShapeDtypeStruct((m, n), jnp.bfloat16),
        in_specs=[
            pl.BlockSpec(memory_space=pltpu.VMEM),
            pl.BlockSpec(memory_space=pltpu.VMEM),
            pl.BlockSpec(memory_space=pltpu.SMEM),
            pl.BlockSpec(memory_space=pltpu.SMEM),
        ],
        out_specs=pl.BlockSpec(memory_space=pl.ANY),
        scratch_shapes=[
            pltpu.VMEM((N_STREAMS, 2, mc, qn), jnp.bfloat16),
            pltpu.SemaphoreType.DMA((N_STREAMS,)),
            pltpu.SemaphoreType.DMA((N_STREAMS,)),
            pltpu.SemaphoreType.REGULAR((N_STREAMS,)),
            pltpu.SemaphoreType.DMA((N_STREAMS,)),
        ],
        compiler_params=pltpu.CompilerParams(
            collective_id=0, vmem_limit_bytes=100 * 1024 * 1024),
    )(x, w_mat, scale_x, scale_w)
